# Initial kernel scaffold; baseline (speedup 1.0000x reference)
import jax
import jax.numpy as jnp
from jax.experimental import pallas as pl


def _rope_block(ids_ref, cos_ref, sin_ref, t_ref, out_ref):
    n_tok = ids_ref.shape[-1]
    n_rows, d = cos_ref.shape
    ids = ids_ref[0, 0, :]
    row_iota = jax.lax.broadcasted_iota(jnp.int32, (n_tok, n_rows), 1)
    onehot = (ids[:, None] == row_iota).astype(jnp.float32)
    lane = jax.lax.broadcasted_iota(jnp.int32, (n_rows, d), 1)
    sin_signed = jnp.where(lane % 2 == 0, -sin_ref[...], sin_ref[...])
    cos_g = jnp.dot(onehot, cos_ref[...],
                    preferred_element_type=jnp.float32)
    sin_g = jnp.dot(onehot, sin_signed,
                    preferred_element_type=jnp.float32)
    tb = t_ref[0]
    h = tb.shape[0]
    tr = tb.reshape(h, n_tok, d // 2, 2)
    swap = jnp.stack((tr[..., 1], tr[..., 0]), axis=-1).reshape(h, n_tok, d)
    out_ref[0] = tb * cos_g[None] + swap * sin_g[None]


def kernel(t, rope_ids, freqs_cos, freqs_sin):
    b, h, n, d = t.shape
    r = freqs_cos.shape[0]
    ids3 = rope_ids.reshape(b, 1, n)
    return pl.pallas_call(
        _rope_block,
        grid=(b,),
        in_specs=[
            pl.BlockSpec((1, 1, n), lambda i: (i, 0, 0)),
            pl.BlockSpec((r, d), lambda i: (0, 0)),
            pl.BlockSpec((r, d), lambda i: (0, 0)),
            pl.BlockSpec((1, h, n, d), lambda i: (i, 0, 0, 0)),
        ],
        out_specs=pl.BlockSpec((1, h, n, d), lambda i: (i, 0, 0, 0)),
        out_shape=jax.ShapeDtypeStruct((b, h, n, d), t.dtype),
    )(ids3, freqs_cos, freqs_sin, t)


# speedup vs baseline: 2.7762x; 2.7762x over previous
import jax
import jax.numpy as jnp
from jax.experimental import pallas as pl


def _rope_block(ids_ref, cos_ref, sin_ref, t_ref, out_ref):
    n_tok = ids_ref.shape[-1]
    n_rows, d = cos_ref.shape
    ids = ids_ref[0, 0, :]
    row_iota = jax.lax.broadcasted_iota(jnp.int32, (n_tok, n_rows), 1)
    onehot = (ids[:, None] == row_iota).astype(jnp.float32)
    cos_g = jnp.dot(onehot, cos_ref[...],
                    preferred_element_type=jnp.float32)
    sin_g = jnp.dot(onehot, sin_ref[...],
                    preferred_element_type=jnp.float32)
    rowm = jax.lax.broadcasted_iota(jnp.int32, (d, d), 0)
    colm = jax.lax.broadcasted_iota(jnp.int32, (d, d), 1)
    m = jnp.where((rowm == colm + 1) & (colm % 2 == 0), -1.0,
                  jnp.where((rowm == colm - 1) & (colm % 2 == 1), 1.0, 0.0))
    tb = t_ref[0]
    h = tb.shape[0]
    t2 = tb.reshape(h * n_tok, d)
    rot = jnp.dot(t2, m, preferred_element_type=jnp.float32).reshape(h, n_tok, d)
    out_ref[0] = tb * cos_g[None] + rot * sin_g[None]


def kernel(t, rope_ids, freqs_cos, freqs_sin):
    b, h, n, d = t.shape
    r = freqs_cos.shape[0]
    ids3 = rope_ids.reshape(b, 1, n)
    return pl.pallas_call(
        _rope_block,
        grid=(b,),
        in_specs=[
            pl.BlockSpec((1, 1, n), lambda i: (i, 0, 0)),
            pl.BlockSpec((r, d), lambda i: (0, 0)),
            pl.BlockSpec((r, d), lambda i: (0, 0)),
            pl.BlockSpec((1, h, n, d), lambda i: (i, 0, 0, 0)),
        ],
        out_specs=pl.BlockSpec((1, h, n, d), lambda i: (i, 0, 0, 0)),
        out_shape=jax.ShapeDtypeStruct((b, h, n, d), t.dtype),
    )(ids3, freqs_cos, freqs_sin, t)
